# Initial kernel scaffold; baseline (speedup 1.0000x reference)
import jax
import jax.numpy as jnp
from jax.experimental import pallas as pl


def kernel(x, pos_embed, rel_pos_coords, gamma, beta, Wa1, ba1, Wa2, ba2, Wf, bf):
    raise NotImplementedError("write your pallas kernel here")


# speedup vs baseline: 19.3283x; 19.3283x over previous
import functools

import jax
import jax.numpy as jnp
from jax import lax
from jax.experimental import pallas as pl
from jax.experimental.pallas import tpu as pltpu
from jax.experimental.pallas import tpu_sc as plsc

KNB = 8


_PREP_ROWS = 1024


def _prep_body(x_ref, pe_ref, gamma_ref, beta_ref, wa1_ref, ba1_ref, wf_ref,
               bf_ref, xp_ref, aself_ref, fself_ref):
    x = x_ref[...]
    mean = jnp.mean(x, axis=-1, keepdims=True)
    var = jnp.mean((x - mean) ** 2, axis=-1, keepdims=True)
    xn = (x - mean) / jnp.sqrt(var + 1e-5) * gamma_ref[...] + beta_ref[...]
    xp = xn + pe_ref[...]
    xp_ref[...] = xp
    wa1_top = wa1_ref[0:128, :]
    aself_ref[...] = (
        jnp.dot(xp, wa1_top, preferred_element_type=jnp.float32) + ba1_ref[...]
    )
    wf_diff = wf_ref[0:128, :] - wf_ref[128:256, :]
    fself_ref[...] = (
        jnp.dot(xp, wf_diff, preferred_element_type=jnp.float32) + bf_ref[...]
    )


def _prep(x2d, pe2d, gamma, beta, wa1, ba1, wf, bf):
    n = x2d.shape[0]
    grid = n // _PREP_ROWS
    row_spec = pl.BlockSpec((_PREP_ROWS, 128), lambda i: (i, 0))
    full = lambda shape: pl.BlockSpec(shape, lambda i: tuple(0 for _ in shape))
    return pl.pallas_call(
        _prep_body,
        grid=(grid,),
        in_specs=[
            row_spec, row_spec,
            full((1, 128)), full((1, 128)),
            full((256, 128)), full((1, 128)),
            full((256, 128)), full((1, 128)),
        ],
        out_specs=[row_spec, row_spec, row_spec],
        out_shape=[jax.ShapeDtypeStruct((n, 128), jnp.float32)] * 3,
    )(x2d, pe2d, gamma, beta, wa1, ba1, wf, bf)


_KNN_ROWS = 256


def _knn_body(cr_ref, ct_ref, idx_ref):
    b = pl.program_id(0)
    i = pl.program_id(1)
    npts = ct_ref.shape[2]
    x_r = cr_ref[0]
    x_ct = ct_ref[0]
    inner = jnp.dot(x_r, x_ct, preferred_element_type=jnp.float32)
    xx_r = jnp.sum(x_r * x_r, axis=1, keepdims=True)
    xx_c = jnp.sum(x_ct * x_ct, axis=0, keepdims=True)
    d = (xx_r + xx_c) + (-2.0) * inner
    rows = lax.broadcasted_iota(jnp.int32, d.shape, 0) + i * _KNN_ROWS
    cols = lax.broadcasted_iota(jnp.int32, d.shape, 1)
    d = jnp.where(rows == cols, jnp.float32(1e9), d)
    base = b * npts
    for t in range(KNB):
        m = jnp.min(d, axis=1, keepdims=True)
        cand = jnp.where(d == m, cols, jnp.int32(npts))
        j = jnp.min(cand, axis=1, keepdims=True)
        idx_ref[0, :, t:t + 1] = j + base
        d = jnp.where(cols == j, jnp.float32(3e38), d)


def _knn(coords, coords_t):
    bsz, npts, _ = coords.shape
    grid = (bsz, npts // _KNN_ROWS)
    return pl.pallas_call(
        _knn_body,
        grid=grid,
        in_specs=[
            pl.BlockSpec((1, _KNN_ROWS, 3), lambda b, i: (b, i, 0)),
            pl.BlockSpec((1, 3, npts), lambda b, i: (b, 0, 0)),
        ],
        out_specs=pl.BlockSpec((1, _KNN_ROWS, KNB), lambda b, i: (b, i, 0)),
        out_shape=jax.ShapeDtypeStruct((bsz, npts, KNB), jnp.int32),
    )(coords, coords_t)


_GCHUNK = 128


def _gather_rows(table, idx_flat):
    n_idx = idx_flat.shape[0]
    info = plsc.get_sparse_core_info()
    nw = info.num_cores * info.num_subcores
    per_w = n_idx // nw
    chunks = per_w // _GCHUNK
    mesh = plsc.VectorSubcoreMesh(core_axis_name="c", subcore_axis_name="s")

    @functools.partial(
        pl.kernel,
        mesh=mesh,
        out_type=jax.ShapeDtypeStruct((n_idx, 128), jnp.float32),
        scratch_types=[
            pltpu.VMEM((_GCHUNK,), jnp.int32),
            pltpu.VMEM((_GCHUNK, 128), jnp.float32),
            pltpu.SemaphoreType.DMA,
        ],
    )
    def gather_k(table_hbm, idx_hbm, out_hbm, idx_v, rows_v, sem):
        wid = lax.axis_index("s") * info.num_cores + lax.axis_index("c")
        base = wid * per_w

        def body(c, carry):
            off = base + c * _GCHUNK
            pltpu.sync_copy(idx_hbm.at[pl.ds(off, _GCHUNK)], idx_v)
            pltpu.async_copy(table_hbm.at[idx_v], rows_v, sem).wait()
            pltpu.sync_copy(rows_v, out_hbm.at[pl.ds(off, _GCHUNK)])
            return carry

        lax.fori_loop(0, chunks, body, 0)

    return gather_k(table, idx_flat)


_EDGE_ROWS = 512


def _leaky(v):
    return jnp.where(v >= 0, v, 0.2 * v)


def _edge_body(xg_ref, aself_ref, fself_ref, wa1_ref, wf_ref, wa2_ref, ba2_ref,
               out_ref):
    asf = aself_ref[...]
    fsf = fself_ref[...]
    wa1b = wa1_ref[128:256, :]
    wfb = wf_ref[128:256, :]
    wa2 = wa2_ref[...]
    ba2 = ba2_ref[0, 0]
    logits = []
    for k in range(KNB):
        xk = xg_ref[k]
        h = _leaky(asf + jnp.dot(xk, wa1b, preferred_element_type=jnp.float32))
        logits.append(jnp.sum(h * wa2, axis=1, keepdims=True) + ba2)
    m = logits[0]
    for k in range(1, KNB):
        m = jnp.maximum(m, logits[k])
    es = [jnp.exp(l - m) for l in logits]
    s = es[0]
    for k in range(1, KNB):
        s = s + es[k]
    acc = jnp.zeros_like(asf)
    for k in range(KNB):
        xk = xg_ref[k]
        t = _leaky(fsf + jnp.dot(xk, wfb, preferred_element_type=jnp.float32))
        acc = acc + (es[k] / s) * t
    out_ref[...] = acc


def _edge(xg3, aself, fself, wa1, wf, wa2t, ba2):
    n = aself.shape[0]
    grid = n // _EDGE_ROWS
    row_spec = pl.BlockSpec((_EDGE_ROWS, 128), lambda i: (i, 0))
    full = lambda shape: pl.BlockSpec(shape, lambda i: tuple(0 for _ in shape))
    return pl.pallas_call(
        _edge_body,
        grid=(grid,),
        in_specs=[
            pl.BlockSpec((KNB, _EDGE_ROWS, 128), lambda i: (0, i, 0)),
            row_spec, row_spec,
            full((256, 128)), full((256, 128)), full((1, 128)), full((1, 1)),
        ],
        out_specs=row_spec,
        out_shape=jax.ShapeDtypeStruct((n, 128), jnp.float32),
    )(xg3, aself, fself, wa1, wf, wa2t, ba2)


def kernel(x, pos_embed, rel_pos_coords, gamma, beta, Wa1, ba1, Wa2, ba2, Wf,
           bf):
    bsz, npts, ch = x.shape
    n = bsz * npts
    x2d = x.reshape(n, ch)
    pe2d = pos_embed.reshape(n, ch)
    xp, a_self, f_self = _prep(
        x2d, pe2d,
        gamma.reshape(1, ch), beta.reshape(1, ch),
        Wa1, ba1.reshape(1, ch), Wf, bf.reshape(1, ch),
    )
    idx = _knn(rel_pos_coords, jnp.swapaxes(rel_pos_coords, 1, 2))
    idx_flat = jnp.transpose(idx, (2, 0, 1)).reshape(-1)
    xg = _gather_rows(xp, idx_flat)
    xg3 = xg.reshape(KNB, n, ch)
    agg = _edge(xg3, a_self, f_self, Wa1, Wf,
                Wa2.reshape(1, ch), ba2.reshape(1, 1))
    return agg.reshape(bsz, npts, ch)


# speedup vs baseline: 22.7548x; 1.1773x over previous
import functools

import jax
import jax.numpy as jnp
from jax import lax
from jax.experimental import pallas as pl
from jax.experimental.pallas import tpu as pltpu
from jax.experimental.pallas import tpu_sc as plsc

KNB = 8


_PREP_ROWS = 1024


def _prep_body(x_ref, pe_ref, gamma_ref, beta_ref, wa1_ref, ba1_ref, wf_ref,
               bf_ref, xp_ref, aself_ref, fself_ref):
    x = x_ref[...]
    mean = jnp.mean(x, axis=-1, keepdims=True)
    var = jnp.mean((x - mean) ** 2, axis=-1, keepdims=True)
    xn = (x - mean) / jnp.sqrt(var + 1e-5) * gamma_ref[...] + beta_ref[...]
    xp = xn + pe_ref[...]
    xp_ref[...] = xp
    wa1_top = wa1_ref[0:128, :]
    aself_ref[...] = (
        jnp.dot(xp, wa1_top, preferred_element_type=jnp.float32) + ba1_ref[...]
    )
    wf_diff = wf_ref[0:128, :] - wf_ref[128:256, :]
    fself_ref[...] = (
        jnp.dot(xp, wf_diff, preferred_element_type=jnp.float32) + bf_ref[...]
    )


def _prep(x2d, pe2d, gamma, beta, wa1, ba1, wf, bf):
    n = x2d.shape[0]
    grid = n // _PREP_ROWS
    row_spec = pl.BlockSpec((_PREP_ROWS, 128), lambda i: (i, 0))
    full = lambda shape: pl.BlockSpec(shape, lambda i: tuple(0 for _ in shape))
    return pl.pallas_call(
        _prep_body,
        grid=(grid,),
        in_specs=[
            row_spec, row_spec,
            full((1, 128)), full((1, 128)),
            full((256, 128)), full((1, 128)),
            full((256, 128)), full((1, 128)),
        ],
        out_specs=[row_spec, row_spec, row_spec],
        out_shape=[jax.ShapeDtypeStruct((n, 128), jnp.float32)] * 3,
    )(x2d, pe2d, gamma, beta, wa1, ba1, wf, bf)


_KNN_ROWS = 256


def _knn_body(cr_ref, ct_ref, idx_ref):
    b = pl.program_id(0)
    i = pl.program_id(1)
    npts = ct_ref.shape[2]
    x_r = cr_ref[0]
    x_ct = ct_ref[0]
    inner = jnp.dot(x_r, x_ct, preferred_element_type=jnp.float32)
    xx_r = jnp.sum(x_r * x_r, axis=1, keepdims=True)
    xx_c = jnp.sum(x_ct * x_ct, axis=0, keepdims=True)
    d = (xx_r + xx_c) + (-2.0) * inner
    rows = lax.broadcasted_iota(jnp.int32, d.shape, 0) + i * _KNN_ROWS
    cols = lax.broadcasted_iota(jnp.int32, d.shape, 1)
    colsf = cols.astype(jnp.float32)
    d = jnp.where(rows == cols, jnp.float32(1e9), d)
    base = b * npts
    for t in range(KNB):
        m = jnp.min(d, axis=1, keepdims=True)
        eqm = d == m
        jf = jnp.min(jnp.where(eqm, colsf, jnp.float32(npts)),
                     axis=1, keepdims=True)
        idx_ref[0, :, t:t + 1] = jf.astype(jnp.int32) + base
        d = jnp.where(eqm, jnp.float32(3e38), d)


def _knn(coords, coords_t):
    bsz, npts, _ = coords.shape
    grid = (bsz, npts // _KNN_ROWS)
    return pl.pallas_call(
        _knn_body,
        grid=grid,
        in_specs=[
            pl.BlockSpec((1, _KNN_ROWS, 3), lambda b, i: (b, i, 0)),
            pl.BlockSpec((1, 3, npts), lambda b, i: (b, 0, 0)),
        ],
        out_specs=pl.BlockSpec((1, _KNN_ROWS, KNB), lambda b, i: (b, i, 0)),
        out_shape=jax.ShapeDtypeStruct((bsz, npts, KNB), jnp.int32),
    )(coords, coords_t)


_GCHUNK = 128


def _gather_rows(table, idx_flat):
    n_idx = idx_flat.shape[0]
    info = plsc.get_sparse_core_info()
    nw = info.num_cores * info.num_subcores
    per_w = n_idx // nw
    chunks = per_w // _GCHUNK
    mesh = plsc.VectorSubcoreMesh(core_axis_name="c", subcore_axis_name="s")

    @functools.partial(
        pl.kernel,
        mesh=mesh,
        out_type=jax.ShapeDtypeStruct((n_idx, 128), jnp.float32),
        scratch_types=[
            pltpu.VMEM((_GCHUNK,), jnp.int32),
            pltpu.VMEM((_GCHUNK, 128), jnp.float32),
            pltpu.SemaphoreType.DMA,
        ],
    )
    def gather_k(table_hbm, idx_hbm, out_hbm, idx_v, rows_v, sem):
        wid = lax.axis_index("s") * info.num_cores + lax.axis_index("c")
        base = wid * per_w

        def body(c, carry):
            off = base + c * _GCHUNK
            pltpu.sync_copy(idx_hbm.at[pl.ds(off, _GCHUNK)], idx_v)
            pltpu.async_copy(table_hbm.at[idx_v], rows_v, sem).wait()
            pltpu.sync_copy(rows_v, out_hbm.at[pl.ds(off, _GCHUNK)])
            return carry

        lax.fori_loop(0, chunks, body, 0)

    return gather_k(table, idx_flat)


_EDGE_ROWS = 512


def _leaky(v):
    return jnp.where(v >= 0, v, 0.2 * v)


def _edge_body(xg_ref, aself_ref, fself_ref, wa1_ref, wf_ref, wa2_ref, ba2_ref,
               out_ref):
    asf = aself_ref[...]
    fsf = fself_ref[...]
    wa1b = wa1_ref[128:256, :]
    wfb = wf_ref[128:256, :]
    wa2 = wa2_ref[...]
    ba2 = ba2_ref[0, 0]
    logits = []
    for k in range(KNB):
        xk = xg_ref[k]
        h = _leaky(asf + jnp.dot(xk, wa1b, preferred_element_type=jnp.float32))
        logits.append(jnp.sum(h * wa2, axis=1, keepdims=True) + ba2)
    m = logits[0]
    for k in range(1, KNB):
        m = jnp.maximum(m, logits[k])
    es = [jnp.exp(l - m) for l in logits]
    s = es[0]
    for k in range(1, KNB):
        s = s + es[k]
    acc = jnp.zeros_like(asf)
    for k in range(KNB):
        xk = xg_ref[k]
        t = _leaky(fsf + jnp.dot(xk, wfb, preferred_element_type=jnp.float32))
        acc = acc + (es[k] / s) * t
    out_ref[...] = acc


def _edge(xg3, aself, fself, wa1, wf, wa2t, ba2):
    n = aself.shape[0]
    grid = n // _EDGE_ROWS
    row_spec = pl.BlockSpec((_EDGE_ROWS, 128), lambda i: (i, 0))
    full = lambda shape: pl.BlockSpec(shape, lambda i: tuple(0 for _ in shape))
    return pl.pallas_call(
        _edge_body,
        grid=(grid,),
        in_specs=[
            pl.BlockSpec((KNB, _EDGE_ROWS, 128), lambda i: (0, i, 0)),
            row_spec, row_spec,
            full((256, 128)), full((256, 128)), full((1, 128)), full((1, 1)),
        ],
        out_specs=row_spec,
        out_shape=jax.ShapeDtypeStruct((n, 128), jnp.float32),
    )(xg3, aself, fself, wa1, wf, wa2t, ba2)


def kernel(x, pos_embed, rel_pos_coords, gamma, beta, Wa1, ba1, Wa2, ba2, Wf,
           bf):
    bsz, npts, ch = x.shape
    n = bsz * npts
    x2d = x.reshape(n, ch)
    pe2d = pos_embed.reshape(n, ch)
    xp, a_self, f_self = _prep(
        x2d, pe2d,
        gamma.reshape(1, ch), beta.reshape(1, ch),
        Wa1, ba1.reshape(1, ch), Wf, bf.reshape(1, ch),
    )
    idx = _knn(rel_pos_coords, jnp.swapaxes(rel_pos_coords, 1, 2))
    idx_flat = jnp.transpose(idx, (2, 0, 1)).reshape(-1)
    xg = _gather_rows(xp, idx_flat)
    xg3 = xg.reshape(KNB, n, ch)
    agg = _edge(xg3, a_self, f_self, Wa1, Wf,
                Wa2.reshape(1, ch), ba2.reshape(1, 1))
    return agg.reshape(bsz, npts, ch)


# speedup vs baseline: 23.1998x; 1.0196x over previous
import functools

import jax
import jax.numpy as jnp
from jax import lax
from jax.experimental import pallas as pl
from jax.experimental.pallas import tpu as pltpu
from jax.experimental.pallas import tpu_sc as plsc

KNB = 8


_PREP_ROWS = 1024


def _prep_body(x_ref, pe_ref, gamma_ref, beta_ref, wa1_ref, ba1_ref, wf_ref,
               bf_ref, xp_ref, aself_ref, fself_ref):
    x = x_ref[...]
    mean = jnp.mean(x, axis=-1, keepdims=True)
    var = jnp.mean((x - mean) ** 2, axis=-1, keepdims=True)
    xn = (x - mean) / jnp.sqrt(var + 1e-5) * gamma_ref[...] + beta_ref[...]
    xp = xn + pe_ref[...]
    xp_ref[...] = xp
    wa1_top = wa1_ref[0:128, :]
    aself_ref[...] = (
        jnp.dot(xp, wa1_top, preferred_element_type=jnp.float32) + ba1_ref[...]
    )
    wf_diff = wf_ref[0:128, :] - wf_ref[128:256, :]
    fself_ref[...] = (
        jnp.dot(xp, wf_diff, preferred_element_type=jnp.float32) + bf_ref[...]
    )


def _prep(x2d, pe2d, gamma, beta, wa1, ba1, wf, bf):
    n = x2d.shape[0]
    grid = n // _PREP_ROWS
    row_spec = pl.BlockSpec((_PREP_ROWS, 128), lambda i: (i, 0))
    full = lambda shape: pl.BlockSpec(shape, lambda i: tuple(0 for _ in shape))
    return pl.pallas_call(
        _prep_body,
        grid=(grid,),
        in_specs=[
            row_spec, row_spec,
            full((1, 128)), full((1, 128)),
            full((256, 128)), full((1, 128)),
            full((256, 128)), full((1, 128)),
        ],
        out_specs=[row_spec, row_spec, row_spec],
        out_shape=[jax.ShapeDtypeStruct((n, 128), jnp.float32)] * 3,
    )(x2d, pe2d, gamma, beta, wa1, ba1, wf, bf)


_KNN_ROWS = 256


def _knn_body(cr_ref, ct_ref, idx_ref):
    b = pl.program_id(0)
    i = pl.program_id(1)
    npts = ct_ref.shape[2]
    x_r = cr_ref[0]
    x_ct = ct_ref[0]
    inner = jnp.dot(x_r, x_ct, preferred_element_type=jnp.float32)
    xx_r = jnp.sum(x_r * x_r, axis=1, keepdims=True)
    xx_c = jnp.sum(x_ct * x_ct, axis=0, keepdims=True)
    d = (xx_r + xx_c) + (-2.0) * inner
    rows = lax.broadcasted_iota(jnp.int32, d.shape, 0) + i * _KNN_ROWS
    cols = lax.broadcasted_iota(jnp.int32, d.shape, 1)
    colsf = cols.astype(jnp.float32)
    d = jnp.where(rows == cols, jnp.float32(1e9), d)
    base = b * npts
    for t in range(KNB):
        m = jnp.min(d, axis=1, keepdims=True)
        eqm = d == m
        jf = jnp.min(jnp.where(eqm, colsf, jnp.float32(npts)),
                     axis=1, keepdims=True)
        idx_ref[0, :, t:t + 1] = jf.astype(jnp.int32) + base
        d = jnp.where(eqm, jnp.float32(3e38), d)


def _knn(coords, coords_t):
    bsz, npts, _ = coords.shape
    grid = (bsz, npts // _KNN_ROWS)
    return pl.pallas_call(
        _knn_body,
        grid=grid,
        in_specs=[
            pl.BlockSpec((1, _KNN_ROWS, 3), lambda b, i: (b, i, 0)),
            pl.BlockSpec((1, 3, npts), lambda b, i: (b, 0, 0)),
        ],
        out_specs=pl.BlockSpec((1, _KNN_ROWS, KNB), lambda b, i: (b, i, 0)),
        out_shape=jax.ShapeDtypeStruct((bsz, npts, KNB), jnp.int32),
    )(coords, coords_t)


_GCHUNK = 128


def _gather_rows(table, idx_flat):
    n_idx = idx_flat.shape[0]
    info = plsc.get_sparse_core_info()
    nw = info.num_cores * info.num_subcores
    per_w = n_idx // nw
    chunks = per_w // _GCHUNK
    mesh = plsc.VectorSubcoreMesh(core_axis_name="c", subcore_axis_name="s")

    @functools.partial(
        pl.kernel,
        mesh=mesh,
        out_type=jax.ShapeDtypeStruct((n_idx, 128), jnp.float32),
        scratch_types=[
            pltpu.VMEM((_GCHUNK,), jnp.int32),
            pltpu.VMEM((_GCHUNK, 128), jnp.float32),
            pltpu.SemaphoreType.DMA,
        ],
    )
    def gather_k(table_hbm, idx_hbm, out_hbm, idx_v, rows_v, sem):
        wid = lax.axis_index("s") * info.num_cores + lax.axis_index("c")
        base = wid * per_w

        def body(c, carry):
            off = base + c * _GCHUNK
            pltpu.sync_copy(idx_hbm.at[pl.ds(off, _GCHUNK)], idx_v)
            pltpu.async_copy(table_hbm.at[idx_v], rows_v, sem).wait()
            pltpu.sync_copy(rows_v, out_hbm.at[pl.ds(off, _GCHUNK)])
            return carry

        lax.fori_loop(0, chunks, body, 0)

    return gather_k(table, idx_flat)


_EDGE_ROWS = 512


def _leaky(v):
    return jnp.where(v >= 0, v, 0.2 * v)


def _edge_body(xg_ref, aself_ref, fself_ref, wa1_ref, wf_ref, wa2_ref, ba2_ref,
               out_ref):
    asf = aself_ref[...]
    fsf = fself_ref[...]
    wa1b = wa1_ref[128:256, :]
    wfb = wf_ref[128:256, :]
    wa2 = wa2_ref[...]
    ba2 = ba2_ref[0, 0]
    logits = []
    for k in range(KNB):
        xk = xg_ref[:, k, :]
        h = _leaky(asf + jnp.dot(xk, wa1b, preferred_element_type=jnp.float32))
        logits.append(jnp.sum(h * wa2, axis=1, keepdims=True) + ba2)
    m = logits[0]
    for k in range(1, KNB):
        m = jnp.maximum(m, logits[k])
    es = [jnp.exp(l - m) for l in logits]
    s = es[0]
    for k in range(1, KNB):
        s = s + es[k]
    acc = jnp.zeros_like(asf)
    for k in range(KNB):
        xk = xg_ref[:, k, :]
        t = _leaky(fsf + jnp.dot(xk, wfb, preferred_element_type=jnp.float32))
        acc = acc + (es[k] / s) * t
    out_ref[...] = acc


def _edge(xg3, aself, fself, wa1, wf, wa2t, ba2):
    n = aself.shape[0]
    grid = n // _EDGE_ROWS
    row_spec = pl.BlockSpec((_EDGE_ROWS, 128), lambda i: (i, 0))
    full = lambda shape: pl.BlockSpec(shape, lambda i: tuple(0 for _ in shape))
    return pl.pallas_call(
        _edge_body,
        grid=(grid,),
        in_specs=[
            pl.BlockSpec((_EDGE_ROWS, KNB, 128), lambda i: (i, 0, 0)),
            row_spec, row_spec,
            full((256, 128)), full((256, 128)), full((1, 128)), full((1, 1)),
        ],
        out_specs=row_spec,
        out_shape=jax.ShapeDtypeStruct((n, 128), jnp.float32),
    )(xg3, aself, fself, wa1, wf, wa2t, ba2)


def kernel(x, pos_embed, rel_pos_coords, gamma, beta, Wa1, ba1, Wa2, ba2, Wf,
           bf):
    bsz, npts, ch = x.shape
    n = bsz * npts
    x2d = x.reshape(n, ch)
    pe2d = pos_embed.reshape(n, ch)
    xp, a_self, f_self = _prep(
        x2d, pe2d,
        gamma.reshape(1, ch), beta.reshape(1, ch),
        Wa1, ba1.reshape(1, ch), Wf, bf.reshape(1, ch),
    )
    coords_t = jnp.swapaxes(rel_pos_coords, 1, 2)
    wa2t = Wa2.reshape(1, ch)
    ba2r = ba2.reshape(1, 1)
    aggs = []
    for b in range(bsz):
        idx_b = _knn(rel_pos_coords[b:b + 1], coords_t[b:b + 1])
        idx_flat = idx_b.reshape(-1) + b * npts
        xg3 = _gather_rows(xp, idx_flat).reshape(npts, KNB, ch)
        lo = b * npts
        aggs.append(_edge(xg3, a_self[lo:lo + npts], f_self[lo:lo + npts],
                          Wa1, Wf, wa2t, ba2r))
    return jnp.concatenate(aggs).reshape(bsz, npts, ch)
